# baseline (speedup 1.0000x reference)
import functools

import jax
import jax.numpy as jnp
from jax import lax
from jax.experimental import pallas as pl
from jax.experimental.pallas import tpu as pltpu
from jax.experimental.pallas import tpu_sc as plsc

NUM_CLASSES = 8
THRESHOLD = 0.5
L = 16
NC = 2
NS = 16
NW = NC * NS
TR = 128


@functools.lru_cache(maxsize=None)
def _build(n_rows: int):
    rows_per_w = n_rows // NW
    chunk = 4096
    n_chunks = rows_per_w // chunk
    blocks = chunk // TR

    mesh = plsc.VectorSubcoreMesh(core_axis_name="c", subcore_axis_name="s")

    @functools.partial(
        pl.kernel,
        out_type=jax.ShapeDtypeStruct((NW, 64), jnp.int32),
        mesh=mesh,
        compiler_params=pltpu.CompilerParams(
            needs_layout_passes=False, use_tc_tiling_on_sc=False),
        scratch_types=[
            pltpu.VMEM((chunk * NUM_CLASSES,), jnp.float32),
            pltpu.VMEM((chunk * NUM_CLASSES,), jnp.float32),
            pltpu.VMEM((chunk,), jnp.int32),
            pltpu.VMEM((chunk,), jnp.int32),
            pltpu.VMEM((L * 64,), jnp.int32),
            pltpu.VMEM((64,), jnp.int32),
            pltpu.SemaphoreType.DMA,
            pltpu.SemaphoreType.DMA,
        ],
    )
    def cm_kernel(pred_hbm, true_hbm, out_hbm,
                  pred_a, pred_b, true_a, true_b, hist_v, row_v,
                  sem_a, sem_b):
        wid = lax.axis_index("s") * NC + lax.axis_index("c")
        base = wid * rows_per_w
        lane = jnp.arange(L, dtype=jnp.int32)
        lane64 = lane * 64
        zeros = jnp.zeros((L,), jnp.int32)
        ones = jnp.ones((L,), jnp.int32)

        for i in range(64):
            hist_v[pl.ds(i * L, L)] = zeros

        def copies(ci, pred_v, true_v, sem):
            start = base + ci * chunk
            cp = pltpu.make_async_copy(
                pred_hbm.at[pl.ds(start * NUM_CLASSES, chunk * NUM_CLASSES)],
                pred_v, sem)
            ct = pltpu.make_async_copy(
                true_hbm.at[pl.ds(start, chunk)], true_v, sem)
            return cp, ct

        def compute(pred_v, true_v):
            @plsc.parallel_loop(0, blocks, unroll=8)
            def block_body(b):
                pbase = b * (TR * NUM_CLASSES)
                tbase = b * TR
                for k in range(TR // L):
                    m = pred_v[pl.ds(pbase + TR + k * L, L)]
                    mi = ones
                    for j in range(2, NUM_CLASSES):
                        vj = pred_v[pl.ds(pbase + j * TR + k * L, L)]
                        gt = vj > m
                        m = jnp.where(gt, vj, m)
                        mi = jnp.where(gt, jnp.int32(j), mi)
                    label = jnp.where(m > THRESHOLD, mi, 0)
                    t = true_v[pl.ds(tbase + k * L, L)]
                    idx = lane64 + t * NUM_CLASSES + label
                    plsc.addupdate_scatter(hist_v, [idx], ones)

        cp, ct = copies(0, pred_a, true_a, sem_a)
        cp.start(), ct.start(), cp.wait(), ct.wait()

        last = n_chunks - 1

        def pair_body(i, carry):
            cp1, ct1 = copies(jnp.minimum(2 * i + 1, last), pred_b, true_b,
                              sem_b)
            cp1.start(), ct1.start()
            compute(pred_a, true_a)
            cp1.wait(), ct1.wait()
            cp2, ct2 = copies(jnp.minimum(2 * i + 2, last), pred_a, true_a,
                              sem_a)
            cp2.start(), ct2.start()
            compute(pred_b, true_b)
            cp2.wait(), ct2.wait()
            return carry

        lax.fori_loop(0, n_chunks // 2, pair_body, 0)

        for k in range(4):
            acc = hist_v[pl.ds(k * L, L)]
            for l in range(1, L):
                acc = acc + hist_v[pl.ds(l * 64 + k * L, L)]
            row_v[pl.ds(k * L, L)] = acc
        pltpu.sync_copy(row_v, out_hbm.at[wid])

    return cm_kernel


def kernel(y_pred, y_true):
    C = NUM_CLASSES
    y_true = jnp.reshape(y_true, (-1,)).astype(jnp.int32)
    y_pred = jnp.reshape(y_pred, (-1, C))
    n = y_true.shape[0]
    y_tiles = jnp.reshape(
        jnp.transpose(jnp.reshape(y_pred, (n // TR, TR, C)), (0, 2, 1)),
        (-1,))
    partials = _build(n)(y_tiles, y_true)
    cm = partials.sum(axis=0).reshape(C, C).astype(jnp.int64)
    fps = cm.sum(axis=0) - jnp.diag(cm)
    fns = cm.sum(axis=1) - jnp.diag(cm)
    tps = jnp.diag(cm)
    tns = cm.sum() - (fps + fns + tps)
    return (cm, tps, fps, fns, tns)


# speedup vs baseline: 1.0005x; 1.0005x over previous
import functools

import jax
import jax.numpy as jnp
from jax import lax
from jax.experimental import pallas as pl
from jax.experimental.pallas import tpu as pltpu
from jax.experimental.pallas import tpu_sc as plsc

NUM_CLASSES = 8
THRESHOLD = 0.5
L = 16
NC = 2
NS = 16
NW = NC * NS
TR = 128


@functools.lru_cache(maxsize=None)
def _build(n_rows: int):
    rows_per_w = n_rows // NW
    chunk = 2048
    nbuf = 4
    n_chunks = rows_per_w // chunk
    blocks = chunk // TR

    mesh = plsc.VectorSubcoreMesh(core_axis_name="c", subcore_axis_name="s")

    @functools.partial(
        pl.kernel,
        out_type=jax.ShapeDtypeStruct((NW, 64), jnp.int32),
        mesh=mesh,
        compiler_params=pltpu.CompilerParams(
            needs_layout_passes=False, use_tc_tiling_on_sc=False),
        scratch_types=(
            [pltpu.VMEM((chunk * NUM_CLASSES,), jnp.float32)] * nbuf
            + [pltpu.VMEM((chunk,), jnp.int32)] * nbuf
            + [
                pltpu.VMEM((L * 64,), jnp.int32),
                pltpu.VMEM((64,), jnp.int32),
            ]
            + [pltpu.SemaphoreType.DMA] * nbuf
        ),
    )
    def cm_kernel(pred_hbm, true_hbm, out_hbm, *scratch):
        preds = scratch[:nbuf]
        trues = scratch[nbuf:2 * nbuf]
        hist_v, row_v = scratch[2 * nbuf], scratch[2 * nbuf + 1]
        sems = scratch[2 * nbuf + 2:]
        wid = lax.axis_index("s") * NC + lax.axis_index("c")
        base = wid * rows_per_w
        lane = jnp.arange(L, dtype=jnp.int32)
        lane64 = lane * 64
        zeros = jnp.zeros((L,), jnp.int32)
        ones = jnp.ones((L,), jnp.int32)

        for i in range(64):
            hist_v[pl.ds(i * L, L)] = zeros

        def copies(ci, pred_v, true_v, sem):
            start = base + ci * chunk
            cp = pltpu.make_async_copy(
                pred_hbm.at[pl.ds(start * NUM_CLASSES, chunk * NUM_CLASSES)],
                pred_v, sem)
            ct = pltpu.make_async_copy(
                true_hbm.at[pl.ds(start, chunk)], true_v, sem)
            return cp, ct

        def compute(pred_v, true_v):
            @plsc.parallel_loop(0, blocks, unroll=4)
            def block_body(b):
                pbase = b * (TR * NUM_CLASSES)
                tbase = b * TR
                for k in range(TR // L):
                    m = pred_v[pl.ds(pbase + TR + k * L, L)]
                    mi = ones
                    for j in range(2, NUM_CLASSES):
                        vj = pred_v[pl.ds(pbase + j * TR + k * L, L)]
                        gt = vj > m
                        m = jnp.where(gt, vj, m)
                        mi = jnp.where(gt, jnp.int32(j), mi)
                    label = jnp.where(m > THRESHOLD, mi, 0)
                    t = true_v[pl.ds(tbase + k * L, L)]
                    idx = lane64 + t * NUM_CLASSES + label
                    plsc.addupdate_scatter(hist_v, [idx], ones)

        last = n_chunks - 1

        for b in range(nbuf - 1):
            cp, ct = copies(b, preds[b], trues[b], sems[b])
            cp.start(), ct.start()

        def ring_body(i, carry):
            for b in range(nbuf):
                ci = nbuf * i + b
                cp, ct = copies(ci, preds[b], trues[b], sems[b])
                cp.wait(), ct.wait()
                nb = (b + nbuf - 1) % nbuf
                cp2, ct2 = copies(jnp.minimum(ci + nbuf - 1, last),
                                  preds[nb], trues[nb], sems[nb])
                cp2.start(), ct2.start()
                compute(preds[b], trues[b])
            return carry

        lax.fori_loop(0, n_chunks // nbuf, ring_body, 0)

        for b in range(nbuf - 1):
            cp, ct = copies(last, preds[b], trues[b], sems[b])
            cp.wait(), ct.wait()

        for k in range(4):
            acc = hist_v[pl.ds(k * L, L)]
            for l in range(1, L):
                acc = acc + hist_v[pl.ds(l * 64 + k * L, L)]
            row_v[pl.ds(k * L, L)] = acc
        pltpu.sync_copy(row_v, out_hbm.at[wid])

    return cm_kernel


def kernel(y_pred, y_true):
    C = NUM_CLASSES
    y_true = jnp.reshape(y_true, (-1,)).astype(jnp.int32)
    y_pred = jnp.reshape(y_pred, (-1, C))
    n = y_true.shape[0]
    y_tiles = jnp.reshape(
        jnp.transpose(jnp.reshape(y_pred, (n // TR, TR, C)), (0, 2, 1)),
        (-1,))
    partials = _build(n)(y_tiles, y_true)
    cm = partials.sum(axis=0).reshape(C, C).astype(jnp.int64)
    fps = cm.sum(axis=0) - jnp.diag(cm)
    fns = cm.sum(axis=1) - jnp.diag(cm)
    tps = jnp.diag(cm)
    tns = cm.sum() - (fps + fns + tps)
    return (cm, tps, fps, fns, tns)


# speedup vs baseline: 1.0197x; 1.0191x over previous
import functools

import jax
import jax.numpy as jnp
from jax import lax
from jax.experimental import pallas as pl
from jax.experimental.pallas import tpu as pltpu
from jax.experimental.pallas import tpu_sc as plsc

NUM_CLASSES = 8
THRESHOLD = 0.5
L = 16
NC = 2
NS = 16
NW = NC * NS
TR = 128


@functools.lru_cache(maxsize=None)
def _build(n_rows: int):
    rows_per_w = n_rows // NW
    chunk = 4096
    nbuf = 2
    n_chunks = rows_per_w // chunk
    blocks = chunk // TR

    mesh = plsc.VectorSubcoreMesh(core_axis_name="c", subcore_axis_name="s")

    @functools.partial(
        pl.kernel,
        out_type=jax.ShapeDtypeStruct((NW, 64), jnp.int32),
        mesh=mesh,
        compiler_params=pltpu.CompilerParams(
            needs_layout_passes=False, use_tc_tiling_on_sc=False),
        scratch_types=(
            [pltpu.VMEM((chunk * NUM_CLASSES,), jnp.float32)] * nbuf
            + [pltpu.VMEM((chunk,), jnp.int32)] * nbuf
            + [
                pltpu.VMEM((L * 64,), jnp.int32),
                pltpu.VMEM((64,), jnp.int32),
            ]
            + [pltpu.SemaphoreType.DMA] * nbuf
        ),
    )
    def cm_kernel(pred_hbm, true_hbm, out_hbm, *scratch):
        preds = scratch[:nbuf]
        trues = scratch[nbuf:2 * nbuf]
        hist_v, row_v = scratch[2 * nbuf], scratch[2 * nbuf + 1]
        sems = scratch[2 * nbuf + 2:]
        wid = lax.axis_index("s") * NC + lax.axis_index("c")
        base = wid * rows_per_w
        lane = jnp.arange(L, dtype=jnp.int32)
        lane64 = lane * 64
        zeros = jnp.zeros((L,), jnp.int32)
        ones = jnp.ones((L,), jnp.int32)

        for i in range(64):
            hist_v[pl.ds(i * L, L)] = zeros

        def copies(ci, pred_v, true_v, sem):
            start = base + ci * chunk
            cp = pltpu.make_async_copy(
                pred_hbm.at[pl.ds(start * NUM_CLASSES, chunk * NUM_CLASSES)],
                pred_v, sem)
            ct = pltpu.make_async_copy(
                true_hbm.at[pl.ds(start, chunk)], true_v, sem)
            return cp, ct

        def compute(pred_v, true_v):
            @plsc.parallel_loop(0, blocks, unroll=4)
            def block_body(b):
                pbase = b * (TR * NUM_CLASSES)
                tbase = b * TR
                for k in range(TR // L):
                    m = pred_v[pl.ds(pbase + TR + k * L, L)]
                    mi = ones
                    for j in range(2, NUM_CLASSES):
                        vj = pred_v[pl.ds(pbase + j * TR + k * L, L)]
                        gt = vj > m
                        m = jnp.where(gt, vj, m)
                        mi = jnp.where(gt, jnp.int32(j), mi)
                    label = jnp.where(m > THRESHOLD, mi, 0)
                    t = true_v[pl.ds(tbase + k * L, L)]
                    idx = lane64 + t * NUM_CLASSES + label
                    plsc.addupdate_scatter(hist_v, [idx], ones)

        last = n_chunks - 1

        for b in range(nbuf - 1):
            cp, ct = copies(b, preds[b], trues[b], sems[b])
            cp.start(), ct.start()

        def ring_body(i, carry):
            for b in range(nbuf):
                ci = nbuf * i + b
                cp, ct = copies(ci, preds[b], trues[b], sems[b])
                cp.wait(), ct.wait()
                nb = (b + nbuf - 1) % nbuf
                cp2, ct2 = copies(jnp.minimum(ci + nbuf - 1, last),
                                  preds[nb], trues[nb], sems[nb])
                cp2.start(), ct2.start()
                compute(preds[b], trues[b])
            return carry

        lax.fori_loop(0, n_chunks // nbuf, ring_body, 0)

        for b in range(nbuf - 1):
            cp, ct = copies(last, preds[b], trues[b], sems[b])
            cp.wait(), ct.wait()

        for k in range(4):
            acc = hist_v[pl.ds(k * L, L)]
            for l in range(1, L):
                acc = acc + hist_v[pl.ds(l * 64 + k * L, L)]
            row_v[pl.ds(k * L, L)] = acc
        pltpu.sync_copy(row_v, out_hbm.at[wid])

    return cm_kernel


def kernel(y_pred, y_true):
    C = NUM_CLASSES
    y_true = jnp.reshape(y_true, (-1,)).astype(jnp.int32)
    y_pred = jnp.reshape(y_pred, (-1, C))
    n = y_true.shape[0]
    y_tiles = jnp.reshape(
        jnp.transpose(jnp.reshape(y_pred, (n // TR, TR, C)), (0, 2, 1)),
        (-1,))
    partials = _build(n)(y_tiles, y_true)
    cm = partials.sum(axis=0).reshape(C, C).astype(jnp.int64)
    fps = cm.sum(axis=0) - jnp.diag(cm)
    fns = cm.sum(axis=1) - jnp.diag(cm)
    tps = jnp.diag(cm)
    tns = cm.sum() - (fps + fns + tps)
    return (cm, tps, fps, fns, tns)


# speedup vs baseline: 1.0881x; 1.0671x over previous
import functools

import jax
import jax.numpy as jnp
from jax import lax
from jax.experimental import pallas as pl
from jax.experimental.pallas import tpu as pltpu
from jax.experimental.pallas import tpu_sc as plsc

NUM_CLASSES = 8
THRESHOLD = 0.5
L = 16
NC = 2
NS = 16
NW = NC * NS
TR = 128


@functools.lru_cache(maxsize=None)
def _build(n_rows: int):
    rows_per_w = n_rows // NW
    chunk = 4096
    nbuf = 2
    n_chunks = rows_per_w // chunk
    blocks = chunk // TR

    mesh = plsc.VectorSubcoreMesh(core_axis_name="c", subcore_axis_name="s")

    @functools.partial(
        pl.kernel,
        out_type=jax.ShapeDtypeStruct((NW, 64), jnp.int32),
        mesh=mesh,
        compiler_params=pltpu.CompilerParams(
            needs_layout_passes=False, use_tc_tiling_on_sc=False),
        scratch_types=(
            [pltpu.VMEM((chunk // TR, NUM_CLASSES - 1, TR), jnp.float32)]
            * nbuf
            + [pltpu.VMEM((chunk,), jnp.int32)] * nbuf
            + [
                pltpu.VMEM((L * 64,), jnp.int32),
                pltpu.VMEM((64,), jnp.int32),
            ]
            + [pltpu.SemaphoreType.DMA] * nbuf
        ),
    )
    def cm_kernel(pred_hbm, true_hbm, out_hbm, *scratch):
        preds = scratch[:nbuf]
        trues = scratch[nbuf:2 * nbuf]
        hist_v, row_v = scratch[2 * nbuf], scratch[2 * nbuf + 1]
        sems = scratch[2 * nbuf + 2:]
        wid = lax.axis_index("s") * NC + lax.axis_index("c")
        base = wid * rows_per_w
        lane = jnp.arange(L, dtype=jnp.int32)
        lane64 = lane * 64
        zeros = jnp.zeros((L,), jnp.int32)
        ones = jnp.ones((L,), jnp.int32)

        for i in range(64):
            hist_v[pl.ds(i * L, L)] = zeros

        def copies(ci, pred_v, true_v, sem):
            start = base + ci * chunk
            cp = pltpu.make_async_copy(
                pred_hbm.at[pl.ds(start // TR, chunk // TR),
                            pl.ds(1, NUM_CLASSES - 1)],
                pred_v, sem)
            ct = pltpu.make_async_copy(
                true_hbm.at[pl.ds(start, chunk)], true_v, sem)
            return cp, ct

        def compute(pred_v, true_v):
            @plsc.parallel_loop(0, blocks, unroll=4)
            def block_body(b):
                tbase = b * TR
                for k in range(TR // L):
                    m = pred_v[b, 0, pl.ds(k * L, L)]
                    mi = ones
                    for j in range(2, NUM_CLASSES):
                        vj = pred_v[b, j - 1, pl.ds(k * L, L)]
                        gt = vj > m
                        m = jnp.where(gt, vj, m)
                        mi = jnp.where(gt, jnp.int32(j), mi)
                    label = jnp.where(m > THRESHOLD, mi, 0)
                    t = true_v[pl.ds(tbase + k * L, L)]
                    idx = lane64 + t * NUM_CLASSES + label
                    plsc.addupdate_scatter(hist_v, [idx], ones)

        last = n_chunks - 1

        for b in range(nbuf - 1):
            cp, ct = copies(b, preds[b], trues[b], sems[b])
            cp.start(), ct.start()

        def ring_body(i, carry):
            for b in range(nbuf):
                ci = nbuf * i + b
                cp, ct = copies(ci, preds[b], trues[b], sems[b])
                cp.wait(), ct.wait()
                nb = (b + nbuf - 1) % nbuf
                cp2, ct2 = copies(jnp.minimum(ci + nbuf - 1, last),
                                  preds[nb], trues[nb], sems[nb])
                cp2.start(), ct2.start()
                compute(preds[b], trues[b])
            return carry

        lax.fori_loop(0, n_chunks // nbuf, ring_body, 0)

        for b in range(nbuf - 1):
            cp, ct = copies(last, preds[b], trues[b], sems[b])
            cp.wait(), ct.wait()

        for k in range(4):
            acc = hist_v[pl.ds(k * L, L)]
            for l in range(1, L):
                acc = acc + hist_v[pl.ds(l * 64 + k * L, L)]
            row_v[pl.ds(k * L, L)] = acc
        pltpu.sync_copy(row_v, out_hbm.at[wid])

    return cm_kernel


def kernel(y_pred, y_true):
    C = NUM_CLASSES
    y_true = jnp.reshape(y_true, (-1,)).astype(jnp.int32)
    y_pred = jnp.reshape(y_pred, (-1, C))
    n = y_true.shape[0]
    y_tiles = jnp.transpose(jnp.reshape(y_pred, (n // TR, TR, C)), (0, 2, 1))
    partials = _build(n)(y_tiles, y_true)
    cm = partials.sum(axis=0).reshape(C, C).astype(jnp.int64)
    fps = cm.sum(axis=0) - jnp.diag(cm)
    fns = cm.sum(axis=1) - jnp.diag(cm)
    tps = jnp.diag(cm)
    tns = cm.sum() - (fps + fns + tps)
    return (cm, tps, fps, fns, tns)


# speedup vs baseline: 1.0885x; 1.0003x over previous
import functools

import jax
import jax.numpy as jnp
from jax import lax
from jax.experimental import pallas as pl
from jax.experimental.pallas import tpu as pltpu
from jax.experimental.pallas import tpu_sc as plsc

NUM_CLASSES = 8
THRESHOLD = 0.5
L = 16
NC = 2
NS = 16
NW = NC * NS
TR = 128


@functools.lru_cache(maxsize=None)
def _build(n_rows: int):
    rows_per_w = n_rows // NW
    chunk = 4096
    nbuf = 2
    n_chunks = rows_per_w // chunk
    blocks = chunk // TR

    mesh = plsc.VectorSubcoreMesh(core_axis_name="c", subcore_axis_name="s")

    @functools.partial(
        pl.kernel,
        out_type=jax.ShapeDtypeStruct((NW, 64), jnp.int32),
        mesh=mesh,
        compiler_params=pltpu.CompilerParams(
            needs_layout_passes=False, use_tc_tiling_on_sc=False),
        scratch_types=(
            [pltpu.VMEM((chunk // TR, NUM_CLASSES - 1, TR), jnp.float32)]
            * nbuf
            + [pltpu.VMEM((chunk,), jnp.int32)] * nbuf
            + [
                pltpu.VMEM((L * 64,), jnp.int32),
                pltpu.VMEM((64,), jnp.int32),
            ]
            + [pltpu.SemaphoreType.DMA] * nbuf
        ),
    )
    def cm_kernel(pred_hbm, true_hbm, out_hbm, *scratch):
        preds = scratch[:nbuf]
        trues = scratch[nbuf:2 * nbuf]
        hist_v, row_v = scratch[2 * nbuf], scratch[2 * nbuf + 1]
        sems = scratch[2 * nbuf + 2:]
        wid = lax.axis_index("s") * NC + lax.axis_index("c")
        base = wid * rows_per_w
        lane = jnp.arange(L, dtype=jnp.int32)
        lane64 = lane * 64
        zeros = jnp.zeros((L,), jnp.int32)
        ones = jnp.ones((L,), jnp.int32)

        for i in range(64):
            hist_v[pl.ds(i * L, L)] = zeros

        def copies(ci, pred_v, true_v, sem):
            start = base + ci * chunk
            half = chunk // TR // 2
            cp = pltpu.make_async_copy(
                pred_hbm.at[pl.ds(start // TR, half),
                            pl.ds(1, NUM_CLASSES - 1)],
                pred_v.at[pl.ds(0, half)], sem)
            cq = pltpu.make_async_copy(
                pred_hbm.at[pl.ds(start // TR + half, half),
                            pl.ds(1, NUM_CLASSES - 1)],
                pred_v.at[pl.ds(half, half)], sem)
            ct = pltpu.make_async_copy(
                true_hbm.at[pl.ds(start, chunk)], true_v, sem)

            class Pair:
                def start(self):
                    cp.start(), cq.start()
                def wait(self):
                    cp.wait(), cq.wait()

            return Pair(), ct

        def compute(pred_v, true_v):
            @plsc.parallel_loop(0, blocks, unroll=4)
            def block_body(b):
                tbase = b * TR
                for k in range(TR // L):
                    m = pred_v[b, 0, pl.ds(k * L, L)]
                    mi = ones
                    for j in range(2, NUM_CLASSES):
                        vj = pred_v[b, j - 1, pl.ds(k * L, L)]
                        gt = vj > m
                        m = jnp.where(gt, vj, m)
                        mi = jnp.where(gt, jnp.int32(j), mi)
                    label = jnp.where(m > THRESHOLD, mi, 0)
                    t = true_v[pl.ds(tbase + k * L, L)]
                    idx = lane64 + t * NUM_CLASSES + label
                    plsc.addupdate_scatter(hist_v, [idx], ones)

        last = n_chunks - 1

        for b in range(nbuf - 1):
            cp, ct = copies(b, preds[b], trues[b], sems[b])
            cp.start(), ct.start()

        def ring_body(i, carry):
            for b in range(nbuf):
                ci = nbuf * i + b
                cp, ct = copies(ci, preds[b], trues[b], sems[b])
                cp.wait(), ct.wait()
                nb = (b + nbuf - 1) % nbuf
                cp2, ct2 = copies(jnp.minimum(ci + nbuf - 1, last),
                                  preds[nb], trues[nb], sems[nb])
                cp2.start(), ct2.start()
                compute(preds[b], trues[b])
            return carry

        lax.fori_loop(0, n_chunks // nbuf, ring_body, 0)

        for b in range(nbuf - 1):
            cp, ct = copies(last, preds[b], trues[b], sems[b])
            cp.wait(), ct.wait()

        for k in range(4):
            acc = hist_v[pl.ds(k * L, L)]
            for l in range(1, L):
                acc = acc + hist_v[pl.ds(l * 64 + k * L, L)]
            row_v[pl.ds(k * L, L)] = acc
        pltpu.sync_copy(row_v, out_hbm.at[wid])

    return cm_kernel


def kernel(y_pred, y_true):
    C = NUM_CLASSES
    y_true = jnp.reshape(y_true, (-1,)).astype(jnp.int32)
    y_pred = jnp.reshape(y_pred, (-1, C))
    n = y_true.shape[0]
    y_tiles = jnp.transpose(jnp.reshape(y_pred, (n // TR, TR, C)), (0, 2, 1))
    partials = _build(n)(y_tiles, y_true)
    cm = partials.sum(axis=0).reshape(C, C).astype(jnp.int64)
    fps = cm.sum(axis=0) - jnp.diag(cm)
    fns = cm.sum(axis=1) - jnp.diag(cm)
    tps = jnp.diag(cm)
    tns = cm.sum() - (fps + fns + tps)
    return (cm, tps, fps, fns, tns)
